# Initial kernel scaffold; baseline (speedup 1.0000x reference)
import jax
import jax.numpy as jnp
from jax.experimental import pallas as pl


def kernel(x, edge_index, edge_attr, fc1_w, fc1_b, k1_w, k1_b, k2_w, k2_b, k3_w, k3_b, root, conv_b, fc2_w, fc2_b):
    raise NotImplementedError("write your pallas kernel here")


# speedup vs baseline: 1.1925x; 1.1925x over previous
import functools

import jax
import jax.numpy as jnp
from jax import lax
from jax.experimental import pallas as pl
from jax.experimental.pallas import tpu as pltpu
from jax.experimental.pallas import tpu_sc as plsc

N = 10000
E = 80000
WIDTH = 64
WPAD = 128
KER_W = 1024
KER_IN = 6
DEPTH = 6

NC = 2
NS = 16
NW = NC * NS
CHUNK = 128
CPW = 20
EPAD = NW * CPW * CHUNK
NPAD = 10240
STRIPE = NPAD // NS

_sc_mesh = plsc.VectorSubcoreMesh(core_axis_name="c", subcore_axis_name="s")


@functools.partial(
    pl.kernel,
    out_type=jax.ShapeDtypeStruct((EPAD, WPAD), jnp.float32),
    mesh=_sc_mesh,
    scratch_types=[
        pltpu.VMEM((CPW, CHUNK), jnp.int32),
        pltpu.VMEM((CHUNK, WPAD), jnp.float32),
        pltpu.SemaphoreType.DMA,
    ],
)
def _sc_gather(h_hbm, src_hbm, xj_hbm, idx_v, row_v, sem):
    c = lax.axis_index("c")
    s = lax.axis_index("s")
    wid = s * NC + c
    pltpu.sync_copy(src_hbm.at[wid], idx_v)
    for j in range(CPW):
        pltpu.async_copy(h_hbm.at[idx_v.at[j]], row_v, sem).wait()
        pltpu.sync_copy(row_v, xj_hbm.at[pl.ds(wid * CPW * CHUNK + j * CHUNK, CHUNK)])


@functools.partial(
    pl.kernel,
    out_type=jax.ShapeDtypeStruct((NC, NPAD, WPAD), jnp.float32),
    mesh=_sc_mesh,
    scratch_types=[
        pltpu.VMEM((CPW, CHUNK), jnp.int32),
        pltpu.VMEM((CHUNK, WPAD), jnp.float32),
        pltpu.VMEM_SHARED((NPAD, WPAD), jnp.float32),
        pltpu.SemaphoreType.DMA,
    ],
)
def _sc_scatter(msg_hbm, dst_hbm, zero_hbm, p_hbm, idx_v, buf_v, agg_sh, sem):
    c = lax.axis_index("c")
    s = lax.axis_index("s")
    wid = s * NC + c
    pltpu.sync_copy(dst_hbm.at[wid], idx_v)
    pltpu.sync_copy(zero_hbm.at[pl.ds(s * STRIPE, STRIPE)],
                    agg_sh.at[pl.ds(s * STRIPE, STRIPE)])
    plsc.subcore_barrier()
    for j in range(CPW):
        pltpu.sync_copy(msg_hbm.at[pl.ds(wid * CPW * CHUNK + j * CHUNK, CHUNK)], buf_v)
        pltpu.sync_copy(buf_v, agg_sh.at[idx_v.at[j]], add=True)
    plsc.subcore_barrier()
    pltpu.sync_copy(agg_sh.at[pl.ds(s * STRIPE, STRIPE)],
                    p_hbm.at[c, pl.ds(s * STRIPE, STRIPE)])


MLP_EB = 256
MSG_EB = 512
UPD_NB = 1024


def _mlp_body(ea_ref, k1t_ref, k1b_ref, k2t_ref, k2b_ref, k3t_ref, k3b_ref, w2_ref):
    h1 = jnp.maximum(jnp.dot(ea_ref[...], k1t_ref[...]) + k1b_ref[...], 0.0)
    ker = jnp.maximum(jnp.dot(h1, k2t_ref[...]) + k2b_ref[...], 0.0)
    w2_ref[...] = (jnp.dot(ker, k3t_ref[...]) + k3b_ref[...]).astype(jnp.bfloat16)


_mlp_call = pl.pallas_call(
    _mlp_body,
    grid=(EPAD // MLP_EB,),
    in_specs=[
        pl.BlockSpec((MLP_EB, 8), lambda i: (i, 0)),
        pl.BlockSpec((8, KER_W), lambda i: (0, 0)),
        pl.BlockSpec((1, KER_W), lambda i: (0, 0)),
        pl.BlockSpec((KER_W, KER_W), lambda i: (0, 0)),
        pl.BlockSpec((1, KER_W), lambda i: (0, 0)),
        pl.BlockSpec((KER_W, WIDTH * WIDTH), lambda i: (0, 0)),
        pl.BlockSpec((1, WIDTH * WIDTH), lambda i: (0, 0)),
    ],
    out_specs=pl.BlockSpec((MLP_EB, WIDTH * WIDTH), lambda i: (i, 0)),
    out_shape=jax.ShapeDtypeStruct((EPAD, WIDTH * WIDTH), jnp.bfloat16),
)


def _msg_body(xj_ref, w2_ref, msg_ref):
    xj = xj_ref[...].astype(jnp.bfloat16).astype(jnp.float32)
    w2 = w2_ref[...].astype(jnp.float32)
    acc = jnp.zeros((MSG_EB, WPAD), jnp.float32)
    lane = lax.broadcasted_iota(jnp.int32, (MSG_EB, WPAD), 1)
    lo = lane < WIDTH
    for i2 in range(WIDTH // 2):
        m = jnp.where(lo, xj[:, 2 * i2:2 * i2 + 1], xj[:, 2 * i2 + 1:2 * i2 + 2])
        acc = acc + w2[:, i2 * WPAD:(i2 + 1) * WPAD] * m
    msg_ref[...] = acc


_msg_call = pl.pallas_call(
    _msg_body,
    grid=(EPAD // MSG_EB,),
    in_specs=[
        pl.BlockSpec((MSG_EB, WPAD), lambda i: (i, 0)),
        pl.BlockSpec((MSG_EB, WIDTH * WIDTH), lambda i: (i, 0)),
    ],
    out_specs=pl.BlockSpec((MSG_EB, WPAD), lambda i: (i, 0)),
    out_shape=jax.ShapeDtypeStruct((EPAD, WPAD), jnp.float32),
)


def _fc1deg_body(x_ref, d0_ref, d1_ref, fw_ref, fb_ref, h0_ref, rd_ref):
    deg = d0_ref[...][:, 0:1] + d1_ref[...][:, 0:1]
    rd_ref[...] = 1.0 / jnp.maximum(deg, 1.0)
    h0 = x_ref[...] * fw_ref[...] + fb_ref[...]
    h0_ref[...] = jnp.concatenate([h0, jnp.zeros((UPD_NB, WIDTH), jnp.float32)], axis=1)


_fc1deg_call = pl.pallas_call(
    _fc1deg_body,
    grid=(NPAD // UPD_NB,),
    in_specs=[
        pl.BlockSpec((UPD_NB, 1), lambda i: (i, 0)),
        pl.BlockSpec((UPD_NB, WPAD), lambda i: (i, 0)),
        pl.BlockSpec((UPD_NB, WPAD), lambda i: (i, 0)),
        pl.BlockSpec((1, WIDTH), lambda i: (0, 0)),
        pl.BlockSpec((1, WIDTH), lambda i: (0, 0)),
    ],
    out_specs=[
        pl.BlockSpec((UPD_NB, WPAD), lambda i: (i, 0)),
        pl.BlockSpec((UPD_NB, 1), lambda i: (i, 0)),
    ],
    out_shape=[
        jax.ShapeDtypeStruct((NPAD, WPAD), jnp.float32),
        jax.ShapeDtypeStruct((NPAD, 1), jnp.float32),
    ],
)


def _upd_body(p0_ref, p1_ref, h_ref, rd_ref, root_ref, cb_ref, out_ref):
    q = p0_ref[...] + p1_ref[...]
    agg = (q[:, :WIDTH] + q[:, WIDTH:]) * rd_ref[...]
    hr = jnp.dot(h_ref[...][:, :WIDTH], root_ref[...])
    r = jnp.maximum(agg + hr + cb_ref[...], 0.0)
    out_ref[...] = jnp.concatenate([r, jnp.zeros((UPD_NB, WIDTH), jnp.float32)], axis=1)


_upd_call = pl.pallas_call(
    _upd_body,
    grid=(NPAD // UPD_NB,),
    in_specs=[
        pl.BlockSpec((UPD_NB, WPAD), lambda i: (i, 0)),
        pl.BlockSpec((UPD_NB, WPAD), lambda i: (i, 0)),
        pl.BlockSpec((UPD_NB, WPAD), lambda i: (i, 0)),
        pl.BlockSpec((UPD_NB, 1), lambda i: (i, 0)),
        pl.BlockSpec((WIDTH, WIDTH), lambda i: (0, 0)),
        pl.BlockSpec((1, WIDTH), lambda i: (0, 0)),
    ],
    out_specs=pl.BlockSpec((UPD_NB, WPAD), lambda i: (i, 0)),
    out_shape=jax.ShapeDtypeStruct((NPAD, WPAD), jnp.float32),
)


def _fc2_body(h_ref, fw_ref, fb_ref, o_ref):
    o_ref[...] = (jnp.sum(h_ref[...][:, :WIDTH] * fw_ref[...], axis=1, keepdims=True)
                  + fb_ref[...])


_fc2_call = pl.pallas_call(
    _fc2_body,
    grid=(NPAD // UPD_NB,),
    in_specs=[
        pl.BlockSpec((UPD_NB, WPAD), lambda i: (i, 0)),
        pl.BlockSpec((1, WIDTH), lambda i: (0, 0)),
        pl.BlockSpec((1, 1), lambda i: (0, 0)),
    ],
    out_specs=pl.BlockSpec((UPD_NB, 1), lambda i: (i, 0)),
    out_shape=jax.ShapeDtypeStruct((NPAD, 1), jnp.float32),
)


def kernel(x, edge_index, edge_attr, fc1_w, fc1_b, k1_w, k1_b, k2_w, k2_b,
           k3_w, k3_b, root, conv_b, fc2_w, fc2_b):
    src = edge_index[0]
    dst = edge_index[1]
    srcp = jnp.concatenate(
        [src, jnp.zeros((EPAD - E,), jnp.int32)]).reshape(NW, CPW, CHUNK)
    dstp = jnp.concatenate(
        [dst, jnp.full((EPAD - E,), NPAD - 1, jnp.int32)]).reshape(NW, CPW, CHUNK)
    eap = jnp.pad(edge_attr, ((0, EPAD - E), (0, 8 - KER_IN)))

    cidx = jnp.arange(WIDTH * WIDTH)
    rperm = (2 * (cidx // WPAD) + (cidx % WPAD) // WIDTH) * WIDTH + cidx % WIDTH
    k3t2 = jnp.take(k3_w, rperm, axis=0).T
    k3b2 = jnp.take(k3_b, rperm).reshape(1, WIDTH * WIDTH)

    k1t = jnp.pad(k1_w.T, ((0, 8 - KER_IN), (0, 0)))
    w2 = _mlp_call(eap, k1t, k1_b.reshape(1, KER_W), k2_w.T,
                   k2_b.reshape(1, KER_W), k3t2, k3b2)

    xp = jnp.pad(x, ((0, NPAD - N), (0, 0)))
    zeros_nw = jnp.zeros((NPAD, WPAD), jnp.float32)
    ones_e = jnp.ones((EPAD, WPAD), jnp.float32)

    degp = _sc_scatter(ones_e, dstp, zeros_nw)
    h, rd = _fc1deg_call(xp, degp[0], degp[1], fc1_w.T, fc1_b.reshape(1, WIDTH))

    for _ in range(DEPTH):
        xj = _sc_gather(h, srcp)
        msg = _msg_call(xj, w2)
        p = _sc_scatter(msg, dstp, zeros_nw)
        h = _upd_call(p[0], p[1], h, rd, root, conv_b.reshape(1, WIDTH))

    out = _fc2_call(h, fc2_w, fc2_b.reshape(1, 1))
    return out[:N]


# speedup vs baseline: 1.2517x; 1.0497x over previous
import functools

import jax
import jax.numpy as jnp
from jax import lax
from jax.experimental import pallas as pl
from jax.experimental.pallas import tpu as pltpu
from jax.experimental.pallas import tpu_sc as plsc

N = 10000
E = 80000
WIDTH = 64
WPAD = 128
KER_W = 1024
KER_IN = 6
DEPTH = 6

NC = 2
NS = 16
NW = NC * NS
CHUNK = 128
CPW = 20
EPAD = NW * CPW * CHUNK
NPAD = 10240
STRIPE = NPAD // NS
NBUF = 4
LOOK = 2

_sc_mesh = plsc.VectorSubcoreMesh(core_axis_name="c", subcore_axis_name="s")


@functools.partial(
    pl.kernel,
    out_type=jax.ShapeDtypeStruct((EPAD, WPAD), jnp.float32),
    mesh=_sc_mesh,
    scratch_types=[
        pltpu.VMEM((CPW, CHUNK), jnp.int32),
        pltpu.VMEM((NBUF, CHUNK, WPAD), jnp.float32),
        pltpu.SemaphoreType.DMA,
        pltpu.SemaphoreType.DMA,
    ],
)
def _sc_gather(h_hbm, src_hbm, xj_hbm, idx_v, buf_v, sem_g, sem_o):
    c = lax.axis_index("c")
    s = lax.axis_index("s")
    wid = s * NC + c
    base = wid * CPW * CHUNK
    pltpu.sync_copy(src_hbm.at[wid], idx_v)
    g = [None] * CPW
    o = [None] * CPW

    def _start_out(jj):
        g[jj].wait()
        o[jj] = pltpu.async_copy(
            buf_v.at[jj % NBUF], xj_hbm.at[pl.ds(base + jj * CHUNK, CHUNK)], sem_o)

    for j in range(CPW):
        if j >= NBUF:
            o[j - NBUF].wait()
        g[j] = pltpu.async_copy(h_hbm.at[idx_v.at[j]], buf_v.at[j % NBUF], sem_g)
        if j >= LOOK:
            _start_out(j - LOOK)
    for jj in range(CPW - LOOK, CPW):
        _start_out(jj)
    for jj in range(CPW - NBUF, CPW):
        o[jj].wait()


@functools.partial(
    pl.kernel,
    out_type=jax.ShapeDtypeStruct((NC, NPAD, WPAD), jnp.float32),
    mesh=_sc_mesh,
    scratch_types=[
        pltpu.VMEM((CPW, CHUNK), jnp.int32),
        pltpu.VMEM((CHUNK, WPAD), jnp.float32),
        pltpu.VMEM_SHARED((NPAD, WPAD), jnp.float32),
        pltpu.SemaphoreType.DMA,
    ],
)
def _sc_scatter(msg_hbm, dst_hbm, zero_hbm, p_hbm, idx_v, buf_v, agg_sh, sem):
    c = lax.axis_index("c")
    s = lax.axis_index("s")
    wid = s * NC + c
    base = wid * CPW * CHUNK
    pltpu.sync_copy(dst_hbm.at[wid], idx_v)
    pltpu.sync_copy(zero_hbm.at[pl.ds(s * STRIPE, STRIPE)],
                    agg_sh.at[pl.ds(s * STRIPE, STRIPE)])
    plsc.subcore_barrier()
    for j in range(CPW):
        pltpu.sync_copy(msg_hbm.at[pl.ds(base + j * CHUNK, CHUNK)], buf_v)
        pltpu.sync_copy(buf_v, agg_sh.at[idx_v.at[j]], add=True)
    plsc.subcore_barrier()
    pltpu.sync_copy(agg_sh.at[pl.ds(s * STRIPE, STRIPE)],
                    p_hbm.at[c, pl.ds(s * STRIPE, STRIPE)])


MLP_EB = 256
MSG_EB = 256
UPD_NB = 1024


def _mlp_body(eat_ref, k1_ref, k1b_ref, k2_ref, k2b_ref, k3_ref, k3b_ref, w3_ref):
    h1 = jnp.maximum(jnp.dot(k1_ref[...], eat_ref[...],
                             preferred_element_type=jnp.float32) + k1b_ref[...], 0.0)
    ker = jnp.maximum(jnp.dot(k2_ref[...], h1.astype(jnp.bfloat16),
                              preferred_element_type=jnp.float32) + k2b_ref[...], 0.0)
    w3 = jnp.dot(k3_ref[...], ker.astype(jnp.bfloat16),
                 preferred_element_type=jnp.float32) + k3b_ref[...]
    w3_ref[...] = w3.astype(jnp.bfloat16)


_mlp_call = pl.pallas_call(
    _mlp_body,
    grid=(EPAD // MLP_EB,),
    in_specs=[
        pl.BlockSpec((8, MLP_EB), lambda i: (0, i)),
        pl.BlockSpec((KER_W, 8), lambda i: (0, 0)),
        pl.BlockSpec((KER_W, 1), lambda i: (0, 0)),
        pl.BlockSpec((KER_W, KER_W), lambda i: (0, 0)),
        pl.BlockSpec((KER_W, 1), lambda i: (0, 0)),
        pl.BlockSpec((WIDTH * WIDTH, KER_W), lambda i: (0, 0)),
        pl.BlockSpec((WIDTH * WIDTH, 1), lambda i: (0, 0)),
    ],
    out_specs=pl.BlockSpec((WIDTH * WIDTH, MLP_EB), lambda i: (0, i)),
    out_shape=jax.ShapeDtypeStruct((WIDTH * WIDTH, EPAD), jnp.bfloat16),
)


def _msg_body(xj_ref, w3_ref, msg_ref):
    xjt = xj_ref[...][:, :WIDTH].astype(jnp.bfloat16).astype(jnp.float32).T
    acc = jnp.zeros((WIDTH, MSG_EB), jnp.float32)
    for i in range(WIDTH):
        w = w3_ref[pl.ds(i * WIDTH, WIDTH), :].astype(jnp.float32)
        acc = acc + w * xjt[i:i + 1, :]
    msg_ref[...] = jnp.concatenate(
        [acc.T, jnp.zeros((MSG_EB, WIDTH), jnp.float32)], axis=1)


_msg_call = pl.pallas_call(
    _msg_body,
    grid=(EPAD // MSG_EB,),
    in_specs=[
        pl.BlockSpec((MSG_EB, WPAD), lambda i: (i, 0)),
        pl.BlockSpec((WIDTH * WIDTH, MSG_EB), lambda i: (0, i)),
    ],
    out_specs=pl.BlockSpec((MSG_EB, WPAD), lambda i: (i, 0)),
    out_shape=jax.ShapeDtypeStruct((EPAD, WPAD), jnp.float32),
)


def _fc1deg_body(x_ref, d0_ref, d1_ref, fw_ref, fb_ref, h0_ref, rd_ref):
    deg = d0_ref[...][:, 0:1] + d1_ref[...][:, 0:1]
    rd_ref[...] = 1.0 / jnp.maximum(deg, 1.0)
    h0 = x_ref[...] * fw_ref[...] + fb_ref[...]
    h0_ref[...] = jnp.concatenate([h0, jnp.zeros((UPD_NB, WIDTH), jnp.float32)], axis=1)


_fc1deg_call = pl.pallas_call(
    _fc1deg_body,
    grid=(NPAD // UPD_NB,),
    in_specs=[
        pl.BlockSpec((UPD_NB, 1), lambda i: (i, 0)),
        pl.BlockSpec((UPD_NB, WPAD), lambda i: (i, 0)),
        pl.BlockSpec((UPD_NB, WPAD), lambda i: (i, 0)),
        pl.BlockSpec((1, WIDTH), lambda i: (0, 0)),
        pl.BlockSpec((1, WIDTH), lambda i: (0, 0)),
    ],
    out_specs=[
        pl.BlockSpec((UPD_NB, WPAD), lambda i: (i, 0)),
        pl.BlockSpec((UPD_NB, 1), lambda i: (i, 0)),
    ],
    out_shape=[
        jax.ShapeDtypeStruct((NPAD, WPAD), jnp.float32),
        jax.ShapeDtypeStruct((NPAD, 1), jnp.float32),
    ],
)


def _upd_body(p0_ref, p1_ref, h_ref, rd_ref, root_ref, cb_ref, out_ref):
    q = p0_ref[...] + p1_ref[...]
    agg = q[:, :WIDTH] * rd_ref[...]
    hr = jnp.dot(h_ref[...][:, :WIDTH], root_ref[...])
    r = jnp.maximum(agg + hr + cb_ref[...], 0.0)
    out_ref[...] = jnp.concatenate([r, jnp.zeros((UPD_NB, WIDTH), jnp.float32)], axis=1)


_upd_call = pl.pallas_call(
    _upd_body,
    grid=(NPAD // UPD_NB,),
    in_specs=[
        pl.BlockSpec((UPD_NB, WPAD), lambda i: (i, 0)),
        pl.BlockSpec((UPD_NB, WPAD), lambda i: (i, 0)),
        pl.BlockSpec((UPD_NB, WPAD), lambda i: (i, 0)),
        pl.BlockSpec((UPD_NB, 1), lambda i: (i, 0)),
        pl.BlockSpec((WIDTH, WIDTH), lambda i: (0, 0)),
        pl.BlockSpec((1, WIDTH), lambda i: (0, 0)),
    ],
    out_specs=pl.BlockSpec((UPD_NB, WPAD), lambda i: (i, 0)),
    out_shape=jax.ShapeDtypeStruct((NPAD, WPAD), jnp.float32),
)


def _fc2_body(h_ref, fw_ref, fb_ref, o_ref):
    o_ref[...] = (jnp.sum(h_ref[...][:, :WIDTH] * fw_ref[...], axis=1, keepdims=True)
                  + fb_ref[...])


_fc2_call = pl.pallas_call(
    _fc2_body,
    grid=(NPAD // UPD_NB,),
    in_specs=[
        pl.BlockSpec((UPD_NB, WPAD), lambda i: (i, 0)),
        pl.BlockSpec((1, WIDTH), lambda i: (0, 0)),
        pl.BlockSpec((1, 1), lambda i: (0, 0)),
    ],
    out_specs=pl.BlockSpec((UPD_NB, 1), lambda i: (i, 0)),
    out_shape=jax.ShapeDtypeStruct((NPAD, 1), jnp.float32),
)


def kernel(x, edge_index, edge_attr, fc1_w, fc1_b, k1_w, k1_b, k2_w, k2_b,
           k3_w, k3_b, root, conv_b, fc2_w, fc2_b):
    src = edge_index[0]
    dst = edge_index[1]
    srcp = jnp.concatenate(
        [src, jnp.zeros((EPAD - E,), jnp.int32)]).reshape(NW, CPW, CHUNK)
    dstp = jnp.concatenate(
        [dst, jnp.full((EPAD - E,), NPAD - 1, jnp.int32)]).reshape(NW, CPW, CHUNK)
    eat = jnp.pad(edge_attr, ((0, EPAD - E), (0, 8 - KER_IN))).T.astype(jnp.bfloat16)

    k1p = jnp.pad(k1_w, ((0, 0), (0, 8 - KER_IN))).astype(jnp.bfloat16)
    w3 = _mlp_call(eat, k1p, k1_b.reshape(KER_W, 1), k2_w.astype(jnp.bfloat16),
                   k2_b.reshape(KER_W, 1), k3_w.astype(jnp.bfloat16),
                   k3_b.reshape(WIDTH * WIDTH, 1))

    xp = jnp.pad(x, ((0, NPAD - N), (0, 0)))
    zeros_nw = jnp.zeros((NPAD, WPAD), jnp.float32)
    ones_e = jnp.ones((EPAD, WPAD), jnp.float32)

    degp = _sc_scatter(ones_e, dstp, zeros_nw)
    h, rd = _fc1deg_call(xp, degp[0], degp[1], fc1_w.T, fc1_b.reshape(1, WIDTH))

    for _ in range(DEPTH):
        xj = _sc_gather(h, srcp)
        msg = _msg_call(xj, w3)
        p = _sc_scatter(msg, dstp, zeros_nw)
        h = _upd_call(p[0], p[1], h, rd, root, conv_b.reshape(1, WIDTH))

    out = _fc2_call(h, fc2_w, fc2_b.reshape(1, 1))
    return out[:N]


# speedup vs baseline: 1.3525x; 1.0805x over previous
import functools

import jax
import jax.numpy as jnp
from jax import lax
from jax.experimental import pallas as pl
from jax.experimental.pallas import tpu as pltpu
from jax.experimental.pallas import tpu_sc as plsc

N = 10000
E = 80000
WIDTH = 64
WPAD = 128
KER_W = 1024
KER_IN = 6
DEPTH = 6

NC = 2
NS = 16
NW = NC * NS
CHUNK = 128
CPW = 20
EPAD = NW * CPW * CHUNK
NPAD = 10240
STRIPE = NPAD // NS
NBUF = 4
LOOK = 2

_sc_mesh = plsc.VectorSubcoreMesh(core_axis_name="c", subcore_axis_name="s")


@functools.partial(
    pl.kernel,
    out_type=jax.ShapeDtypeStruct((EPAD, WPAD), jnp.float32),
    mesh=_sc_mesh,
    scratch_types=[
        pltpu.VMEM((CPW, CHUNK), jnp.int32),
        pltpu.VMEM((NBUF, CHUNK, WPAD), jnp.float32),
        pltpu.SemaphoreType.DMA,
        pltpu.SemaphoreType.DMA,
    ],
)
def _sc_gather(h_hbm, src_hbm, xj_hbm, idx_v, buf_v, sem_g, sem_o):
    c = lax.axis_index("c")
    s = lax.axis_index("s")
    wid = s * NC + c
    base = wid * CPW * CHUNK
    pltpu.sync_copy(src_hbm.at[wid], idx_v)
    g = [None] * CPW
    o = [None] * CPW

    def _start_out(jj):
        g[jj].wait()
        o[jj] = pltpu.async_copy(
            buf_v.at[jj % NBUF], xj_hbm.at[pl.ds(base + jj * CHUNK, CHUNK)], sem_o)

    for j in range(CPW):
        if j >= NBUF:
            o[j - NBUF].wait()
        g[j] = pltpu.async_copy(h_hbm.at[idx_v.at[j]], buf_v.at[j % NBUF], sem_g)
        if j >= LOOK:
            _start_out(j - LOOK)
    for jj in range(CPW - LOOK, CPW):
        _start_out(jj)
    for jj in range(CPW - NBUF, CPW):
        o[jj].wait()


@functools.partial(
    pl.kernel,
    out_type=jax.ShapeDtypeStruct((NC, NPAD, WPAD), jnp.float32),
    mesh=_sc_mesh,
    scratch_types=[
        pltpu.VMEM((CPW, CHUNK), jnp.int32),
        pltpu.VMEM((CHUNK, WPAD), jnp.float32),
        pltpu.VMEM_SHARED((NPAD, WPAD), jnp.float32),
        pltpu.SemaphoreType.DMA,
    ],
)
def _sc_scatter(msg_hbm, dst_hbm, zero_hbm, p_hbm, idx_v, buf_v, agg_sh, sem):
    c = lax.axis_index("c")
    s = lax.axis_index("s")
    wid = s * NC + c
    base = wid * CPW * CHUNK
    pltpu.sync_copy(dst_hbm.at[wid], idx_v)
    pltpu.sync_copy(zero_hbm.at[pl.ds(s * STRIPE, STRIPE)],
                    agg_sh.at[pl.ds(s * STRIPE, STRIPE)])
    plsc.subcore_barrier()
    for j in range(CPW):
        pltpu.sync_copy(msg_hbm.at[pl.ds(base + j * CHUNK, CHUNK)], buf_v)
        pltpu.sync_copy(buf_v, agg_sh.at[idx_v.at[j]], add=True)
    plsc.subcore_barrier()
    pltpu.sync_copy(agg_sh.at[pl.ds(s * STRIPE, STRIPE)],
                    p_hbm.at[c, pl.ds(s * STRIPE, STRIPE)])


MLP_EB = 512
MSG_EB = 256
UPD_NB = 1024


def _mlp_body(eat_ref, k1_ref, k1b_ref, k2_ref, k2b_ref, k3_ref, k3b_ref, w3_ref):
    h1 = jnp.maximum(jnp.dot(k1_ref[...], eat_ref[...],
                             preferred_element_type=jnp.float32) + k1b_ref[...], 0.0)
    ker = jnp.maximum(jnp.dot(k2_ref[...], h1.astype(jnp.bfloat16),
                              preferred_element_type=jnp.float32) + k2b_ref[...], 0.0)
    w3 = jnp.dot(k3_ref[...], ker.astype(jnp.bfloat16),
                 preferred_element_type=jnp.float32) + k3b_ref[...]
    w3_ref[...] = w3.astype(jnp.bfloat16)


_mlp_call = pl.pallas_call(
    _mlp_body,
    grid=(EPAD // MLP_EB,),
    in_specs=[
        pl.BlockSpec((8, MLP_EB), lambda i: (0, i)),
        pl.BlockSpec((KER_W, 8), lambda i: (0, 0)),
        pl.BlockSpec((KER_W, 1), lambda i: (0, 0)),
        pl.BlockSpec((KER_W, KER_W), lambda i: (0, 0)),
        pl.BlockSpec((KER_W, 1), lambda i: (0, 0)),
        pl.BlockSpec((WIDTH * WIDTH, KER_W), lambda i: (0, 0)),
        pl.BlockSpec((WIDTH * WIDTH, 1), lambda i: (0, 0)),
    ],
    out_specs=pl.BlockSpec((WIDTH * WIDTH, MLP_EB), lambda i: (0, i)),
    out_shape=jax.ShapeDtypeStruct((WIDTH * WIDTH, EPAD), jnp.bfloat16),
)


def _msg_body(xj_ref, w3_ref, msg_ref):
    xjt = xj_ref[...][:, :WIDTH].astype(jnp.bfloat16).astype(jnp.float32).T
    acc = jnp.zeros((WIDTH, MSG_EB), jnp.float32)
    for i in range(WIDTH):
        w = w3_ref[pl.ds(i * WIDTH, WIDTH), :].astype(jnp.float32)
        acc = acc + w * xjt[i:i + 1, :]
    msg_ref[...] = jnp.concatenate(
        [acc.T, jnp.zeros((MSG_EB, WIDTH), jnp.float32)], axis=1)


_msg_call = pl.pallas_call(
    _msg_body,
    grid=(EPAD // MSG_EB,),
    in_specs=[
        pl.BlockSpec((MSG_EB, WPAD), lambda i: (i, 0)),
        pl.BlockSpec((WIDTH * WIDTH, MSG_EB), lambda i: (0, i)),
    ],
    out_specs=pl.BlockSpec((MSG_EB, WPAD), lambda i: (i, 0)),
    out_shape=jax.ShapeDtypeStruct((EPAD, WPAD), jnp.float32),
)


def _fc1deg_body(x_ref, d0_ref, d1_ref, fw_ref, fb_ref, h0_ref, rd_ref):
    deg = d0_ref[...][:, 0:1] + d1_ref[...][:, 0:1]
    rd_ref[...] = 1.0 / jnp.maximum(deg, 1.0)
    xb = x_ref[...].astype(jnp.bfloat16).astype(jnp.float32)
    fwb = fw_ref[...].astype(jnp.bfloat16).astype(jnp.float32)
    h0 = xb * fwb + fb_ref[...]
    h0_ref[...] = jnp.concatenate([h0, jnp.zeros((UPD_NB, WIDTH), jnp.float32)], axis=1)


_fc1deg_call = pl.pallas_call(
    _fc1deg_body,
    grid=(NPAD // UPD_NB,),
    in_specs=[
        pl.BlockSpec((UPD_NB, 1), lambda i: (i, 0)),
        pl.BlockSpec((UPD_NB, WPAD), lambda i: (i, 0)),
        pl.BlockSpec((UPD_NB, WPAD), lambda i: (i, 0)),
        pl.BlockSpec((1, WIDTH), lambda i: (0, 0)),
        pl.BlockSpec((1, WIDTH), lambda i: (0, 0)),
    ],
    out_specs=[
        pl.BlockSpec((UPD_NB, WPAD), lambda i: (i, 0)),
        pl.BlockSpec((UPD_NB, 1), lambda i: (i, 0)),
    ],
    out_shape=[
        jax.ShapeDtypeStruct((NPAD, WPAD), jnp.float32),
        jax.ShapeDtypeStruct((NPAD, 1), jnp.float32),
    ],
)


def _upd_body(p0_ref, p1_ref, h_ref, rd_ref, root_ref, cb_ref, out_ref):
    q = p0_ref[...] + p1_ref[...]
    agg = q[:, :WIDTH] * rd_ref[...]
    hr = jnp.dot(h_ref[...][:, :WIDTH], root_ref[...])
    r = jnp.maximum(agg + hr + cb_ref[...], 0.0)
    out_ref[...] = jnp.concatenate([r, jnp.zeros((UPD_NB, WIDTH), jnp.float32)], axis=1)


_upd_call = pl.pallas_call(
    _upd_body,
    grid=(NPAD // UPD_NB,),
    in_specs=[
        pl.BlockSpec((UPD_NB, WPAD), lambda i: (i, 0)),
        pl.BlockSpec((UPD_NB, WPAD), lambda i: (i, 0)),
        pl.BlockSpec((UPD_NB, WPAD), lambda i: (i, 0)),
        pl.BlockSpec((UPD_NB, 1), lambda i: (i, 0)),
        pl.BlockSpec((WIDTH, WIDTH), lambda i: (0, 0)),
        pl.BlockSpec((1, WIDTH), lambda i: (0, 0)),
    ],
    out_specs=pl.BlockSpec((UPD_NB, WPAD), lambda i: (i, 0)),
    out_shape=jax.ShapeDtypeStruct((NPAD, WPAD), jnp.float32),
)


def _fc2_body(h_ref, fw_ref, fb_ref, o_ref):
    o_ref[...] = (jnp.sum(h_ref[...][:, :WIDTH] * fw_ref[...], axis=1, keepdims=True)
                  + fb_ref[...])


_fc2_call = pl.pallas_call(
    _fc2_body,
    grid=(NPAD // UPD_NB,),
    in_specs=[
        pl.BlockSpec((UPD_NB, WPAD), lambda i: (i, 0)),
        pl.BlockSpec((1, WIDTH), lambda i: (0, 0)),
        pl.BlockSpec((1, 1), lambda i: (0, 0)),
    ],
    out_specs=pl.BlockSpec((UPD_NB, 1), lambda i: (i, 0)),
    out_shape=jax.ShapeDtypeStruct((NPAD, 1), jnp.float32),
)


def kernel(x, edge_index, edge_attr, fc1_w, fc1_b, k1_w, k1_b, k2_w, k2_b,
           k3_w, k3_b, root, conv_b, fc2_w, fc2_b):
    src = edge_index[0]
    dst = edge_index[1]
    srcp = jnp.concatenate(
        [src, jnp.zeros((EPAD - E,), jnp.int32)]).reshape(NW, CPW, CHUNK)
    dstp = jnp.concatenate(
        [dst, jnp.full((EPAD - E,), NPAD - 1, jnp.int32)]).reshape(NW, CPW, CHUNK)
    eat = jnp.pad(edge_attr, ((0, EPAD - E), (0, 8 - KER_IN))).T.astype(jnp.bfloat16)

    k1p = jnp.pad(k1_w, ((0, 0), (0, 8 - KER_IN))).astype(jnp.bfloat16)
    w3 = _mlp_call(eat, k1p, k1_b.reshape(KER_W, 1), k2_w.astype(jnp.bfloat16),
                   k2_b.reshape(KER_W, 1), k3_w.astype(jnp.bfloat16),
                   k3_b.reshape(WIDTH * WIDTH, 1))

    xp = jnp.pad(x, ((0, NPAD - N), (0, 0)))
    zeros_nw = jnp.zeros((NPAD, WPAD), jnp.float32)
    ones_e = jnp.ones((EPAD, WPAD), jnp.float32)

    degp = _sc_scatter(ones_e, dstp, zeros_nw)
    h, rd = _fc1deg_call(xp, degp[0], degp[1], fc1_w.T, fc1_b.reshape(1, WIDTH))

    for _ in range(DEPTH):
        xj = _sc_gather(h, srcp)
        msg = _msg_call(xj, w3)
        p = _sc_scatter(msg, dstp, zeros_nw)
        h = _upd_call(p[0], p[1], h, rd, root, conv_b.reshape(1, WIDTH))

    out = _fc2_call(h, fc2_w, fc2_b.reshape(1, 1))
    return out[:N]
